# baseline (speedup 1.0000x reference)
import functools

import jax
import jax.numpy as jnp
from jax import lax
from jax.experimental import pallas as pl
from jax.experimental.pallas import tpu as pltpu
from jax.experimental.pallas import tpu_sc as plsc

_N = 512
_M = 512
_F = 96
_B = 524288
_HID = 64

_NC = 2
_NS = 16
_NW = _NC * _NS
_ROWS_PER_W = _B // _NW
_CHUNK = 128

_TB = 512


def _feat_body(x0_hbm, x1_hbm, x2_hbm, xy_hbm, yz_hbm, xz_hbm, feat_hbm,
               x0v, x1v, x2v, ixy, iyz, bufa, bufb, bufc, sem):
    wid = lax.axis_index("s") * _NC + lax.axis_index("c")
    wbase = wid * _ROWS_PER_W

    cp0 = pltpu.async_copy(x0_hbm.at[pl.ds(wbase, _ROWS_PER_W)], x0v, sem)
    cp1 = pltpu.async_copy(x1_hbm.at[pl.ds(wbase, _ROWS_PER_W)], x1v, sem)
    cp2 = pltpu.async_copy(x2_hbm.at[pl.ds(wbase, _ROWS_PER_W)], x2v, sem)
    cp0.wait()
    cp1.wait()
    cp2.wait()

    @pl.loop(0, _ROWS_PER_W, step=16)
    def _idx(i):
        s = pl.ds(i, 16)
        a0 = x0v[s]
        a1 = x1v[s]
        a2 = x2v[s]
        fi = (a0 * 0.5 + 0.5) * float(_N)
        fj = (a1 * 0.5 + 0.5) * float(_M)
        gi = (a1 * 0.5 + 0.5) * float(_N)
        gj = (a2 * 0.5 + 0.5) * float(_M)
        ii = jnp.clip(fi.astype(jnp.int32), 0, _N - 1)
        jj = jnp.clip(fj.astype(jnp.int32), 0, _M - 1)
        ki = jnp.clip(gi.astype(jnp.int32), 0, _N - 1)
        kj = jnp.clip(gj.astype(jnp.int32), 0, _M - 1)
        ixy[s] = ii * _M + jj
        iyz[s] = ki * _M + kj

    @pl.loop(0, _ROWS_PER_W, step=_CHUNK)
    def _chunk(r0):
        isl = pl.ds(r0, _CHUNK)
        ga = pltpu.async_copy(xy_hbm.at[ixy.at[isl]], bufa, sem)
        gb = pltpu.async_copy(yz_hbm.at[iyz.at[isl]], bufb, sem)
        gc = pltpu.async_copy(xz_hbm.at[iyz.at[isl]], bufc, sem)
        ga.wait()
        gb.wait()
        gc.wait()

        @pl.loop(0, _CHUNK)
        def _mul(r):
            ra = bufa.at[r]
            rb = bufb.at[r]
            rc = bufc.at[r]
            for cc in range(_F // 16):
                sl = pl.ds(cc * 16, 16)
                ra[sl] = ra[sl] * rb[sl] * rc[sl]

        pltpu.sync_copy(bufa, feat_hbm.at[pl.ds(wbase + r0, _CHUNK)])


def _feat(x0, x1, x2, xyf, yzf, xzf):
    mesh = plsc.VectorSubcoreMesh(core_axis_name="c", subcore_axis_name="s")
    k = pl.kernel(
        _feat_body,
        out_type=jax.ShapeDtypeStruct((_B, _F), jnp.float32),
        mesh=mesh,
        compiler_params=pltpu.CompilerParams(use_tc_tiling_on_sc=False),
        scratch_types=[
            pltpu.VMEM((_ROWS_PER_W,), jnp.float32),
            pltpu.VMEM((_ROWS_PER_W,), jnp.float32),
            pltpu.VMEM((_ROWS_PER_W,), jnp.float32),
            pltpu.VMEM((_ROWS_PER_W,), jnp.int32),
            pltpu.VMEM((_ROWS_PER_W,), jnp.int32),
            pltpu.VMEM((_CHUNK, _F), jnp.float32),
            pltpu.VMEM((_CHUNK, _F), jnp.float32),
            pltpu.VMEM((_CHUNK, _F), jnp.float32),
            pltpu.SemaphoreType.DMA,
        ],
    )
    return k(x0, x1, x2, xyf, yzf, xzf)


def _mlp_body(feat_ref, x_ref, d_ref, w1, b1, w2, b2, w3a, w3b, b3, w4, b4,
              w5, b5, c_ref, sig_ref):
    hp = lax.Precision.HIGHEST
    f = feat_ref[...]
    h = jnp.dot(f, w1[...], preferred_element_type=jnp.float32, precision=hp)
    h = jnp.maximum(h + b1[...], 0.0)
    h2 = jnp.dot(h, w2[...], preferred_element_type=jnp.float32, precision=hp)
    h2 = jnp.maximum(h2 + b2[...], 0.0)

    dblk = d_ref[...]
    pieces = [dblk]
    for j in range(4):
        t = dblk * (2.0 ** j)
        pieces.append(jnp.sin(t))
        pieces.append(jnp.cos(t))
    pe = jnp.concatenate(pieces, axis=1)

    z = (jnp.dot(pe, w3a[...], preferred_element_type=jnp.float32, precision=hp)
         + jnp.dot(h2, w3b[...], preferred_element_type=jnp.float32, precision=hp)
         + b3[...])
    z = jnp.maximum(z, 0.0)
    z = jnp.dot(z, w4[...], preferred_element_type=jnp.float32, precision=hp)
    z = jnp.maximum(z + b4[...], 0.0)
    cm = jax.nn.sigmoid(
        jnp.dot(z, w5[...], preferred_element_type=jnp.float32, precision=hp)
        + b5[...])

    xb = x_ref[...]
    m = ((jnp.abs(xb[:, 0:1]) < 1.0)
         & (jnp.abs(xb[:, 1:2]) < 1.0)
         & (jnp.abs(xb[:, 2:3]) < 1.0))
    sig_ref[...] = jnp.where(m, h2[:, 15:16], 0.0)
    c_ref[...] = jnp.where(m, cm, 0.0)


def _row2(i):
    return (i, 0)


def _fixed_spec(a):
    return pl.BlockSpec(a.shape, lambda i: (0,) * a.ndim)


def _mlp(feat, x, d, w1, b1, w2, b2, w3a, w3b, b3, w4, b4, w5, b5):
    weights = [w1, b1, w2, b2, w3a, w3b, b3, w4, b4, w5, b5]
    grid = (_B // _TB,)
    return pl.pallas_call(
        _mlp_body,
        grid=grid,
        in_specs=[
            pl.BlockSpec((_TB, _F), _row2),
            pl.BlockSpec((_TB, 3), _row2),
            pl.BlockSpec((_TB, 3), _row2),
        ] + [_fixed_spec(w) for w in weights],
        out_specs=[
            pl.BlockSpec((_TB, 3), _row2),
            pl.BlockSpec((_TB, 1), _row2),
        ],
        out_shape=[
            jax.ShapeDtypeStruct((_B, 3), jnp.float32),
            jax.ShapeDtypeStruct((_B, 1), jnp.float32),
        ],
    )(feat, x, d, *weights)


def kernel(x, d, xy_plane, yz_plane, xz_plane, W1, b1, W2, b2, W3, b3, W4,
           b4, W5, b5):
    x0 = x[:, 0]
    x1 = x[:, 1]
    x2 = x[:, 2]
    xyf = xy_plane.reshape(_N * _M, _F)
    yzf = yz_plane.reshape(_N * _M, _F)
    xzf = xz_plane.reshape(_N * _M, _F)
    feat = _feat(x0, x1, x2, xyf, yzf, xzf)

    w3a = W3[:27]
    w3b = jnp.concatenate([W3[27:], jnp.zeros((1, _HID), jnp.float32)], axis=0)
    c, sig2 = _mlp(feat, x, d, W1, b1.reshape(1, -1), W2, b2.reshape(1, -1),
                   w3a, w3b, b3.reshape(1, -1), W4, b4.reshape(1, -1), W5,
                   b5.reshape(1, -1))
    return c, sig2[:, 0]


# speedup vs baseline: 4.0093x; 4.0093x over previous
import functools

import jax
import jax.numpy as jnp
from jax import lax
from jax.experimental import pallas as pl
from jax.experimental.pallas import tpu as pltpu
from jax.experimental.pallas import tpu_sc as plsc

_N = 512
_M = 512
_F = 96
_B = 524288
_HID = 64

_NC = 2
_NS = 16
_NW = _NC * _NS
_ROWS_PER_W = _B // _NW
_CHUNK = 128

_TB = 2048


_SUP = 4096
_SUP_CHUNKS = _SUP // _CHUNK


def _feat_body(x0_hbm, x1_hbm, x2_hbm, xy_hbm, yz_hbm, xz_hbm, feat_hbm,
               x0v, x1v, x2v, ixy, iyz, bufs, obuf, gsem, wsem, xsem):
    wid = lax.axis_index("s") * _NC + lax.axis_index("c")
    wbase = wid * _ROWS_PER_W

    def issue_gathers(sbase, g, bset):
        isl = pl.ds(g * _CHUNK, _CHUNK)
        ba, bb, bc = bufs[bset]
        ca = pltpu.async_copy(xy_hbm.at[ixy.at[isl]], ba, gsem)
        cb = pltpu.async_copy(yz_hbm.at[iyz.at[isl]], bb, gsem)
        cc = pltpu.async_copy(xz_hbm.at[iyz.at[isl]], bc, gsem)
        return ca, cb, cc

    def wait_gathers(bset):
        ba, bb, bc = bufs[bset]
        pltpu.make_async_copy(xy_hbm.at[ixy.at[pl.ds(0, _CHUNK)]], ba, gsem).wait()
        pltpu.make_async_copy(yz_hbm.at[iyz.at[pl.ds(0, _CHUNK)]], bb, gsem).wait()
        pltpu.make_async_copy(xz_hbm.at[iyz.at[pl.ds(0, _CHUNK)]], bc, gsem).wait()

    def wait_writeout(ob):
        pltpu.make_async_copy(ob, feat_hbm.at[pl.ds(0, _CHUNK)], wsem).wait()

    @pl.loop(0, _ROWS_PER_W, step=_SUP)
    def _sup(s0):
        sbase = wbase + s0
        cp0 = pltpu.async_copy(x0_hbm.at[pl.ds(sbase, _SUP)], x0v, xsem)
        cp1 = pltpu.async_copy(x1_hbm.at[pl.ds(sbase, _SUP)], x1v, xsem)
        cp2 = pltpu.async_copy(x2_hbm.at[pl.ds(sbase, _SUP)], x2v, xsem)
        cp0.wait()
        cp1.wait()
        cp2.wait()

        @pl.loop(0, _SUP, step=16)
        def _idx(i):
            s = pl.ds(i, 16)
            a0 = x0v[s]
            a1 = x1v[s]
            a2 = x2v[s]
            fi = (a0 * 0.5 + 0.5) * float(_N)
            fj = (a1 * 0.5 + 0.5) * float(_M)
            gi = (a1 * 0.5 + 0.5) * float(_N)
            gj = (a2 * 0.5 + 0.5) * float(_M)
            ii = jnp.clip(fi.astype(jnp.int32), 0, _N - 1)
            jj = jnp.clip(fj.astype(jnp.int32), 0, _M - 1)
            ki = jnp.clip(gi.astype(jnp.int32), 0, _N - 1)
            kj = jnp.clip(gj.astype(jnp.int32), 0, _M - 1)
            ixy[s] = ii * _M + jj
            iyz[s] = ki * _M + kj

        issue_gathers(sbase, 0, 0)

        @pl.loop(0, _SUP_CHUNKS, step=2)
        def _chunk(g0):
            for b in range(2):
                g = g0 + b

                @pl.when(g + 1 < _SUP_CHUNKS)
                def _():
                    issue_gathers(sbase, g + 1, (b + 1) % 2)

                wait_gathers(b)

                @pl.when(g >= 2)
                def _():
                    wait_writeout(obuf[b])

                ba, bb, bc = bufs[b]
                ob = obuf[b]

                @pl.loop(0, _CHUNK)
                def _mul(r):
                    ra = ba.at[r]
                    rb = bb.at[r]
                    rc = bc.at[r]
                    ro = ob.at[r]
                    for col in range(_F // 16):
                        sl = pl.ds(col * 16, 16)
                        ro[sl] = ra[sl] * rb[sl] * rc[sl]

                pltpu.async_copy(
                    ob, feat_hbm.at[pl.ds(sbase + g * _CHUNK, _CHUNK)], wsem)

        wait_writeout(obuf[0])
        wait_writeout(obuf[1])


def _feat(x0, x1, x2, xyf, yzf, xzf):
    mesh = plsc.VectorSubcoreMesh(core_axis_name="c", subcore_axis_name="s")
    buf_t = pltpu.VMEM((_CHUNK, _F), jnp.float32)
    k = pl.kernel(
        _feat_body,
        out_type=jax.ShapeDtypeStruct((_B, _F), jnp.float32),
        mesh=mesh,
        compiler_params=pltpu.CompilerParams(use_tc_tiling_on_sc=False),
        scratch_types=[
            pltpu.VMEM((_SUP,), jnp.float32),
            pltpu.VMEM((_SUP,), jnp.float32),
            pltpu.VMEM((_SUP,), jnp.float32),
            pltpu.VMEM((_SUP,), jnp.int32),
            pltpu.VMEM((_SUP,), jnp.int32),
            ((buf_t, buf_t, buf_t), (buf_t, buf_t, buf_t)),
            (buf_t, buf_t),
            pltpu.SemaphoreType.DMA,
            pltpu.SemaphoreType.DMA,
            pltpu.SemaphoreType.DMA,
        ],
    )
    return k(x0, x1, x2, xyf, yzf, xzf)


def _mlp_body(feat_ref, xt_ref, dt_ref, w1t, b1, w2t, b2, w3t, b3, w4t, b4,
              w5t, b5, ct_ref, sigt_ref):
    hp = lax.Precision.DEFAULT
    f = feat_ref[...]
    h = lax.dot_general(w1t[...], f, (((1,), (1,)), ((), ())),
                        preferred_element_type=jnp.float32, precision=hp)
    h = jnp.maximum(h + b1[...], 0.0)
    h2 = jnp.dot(w2t[...], h, preferred_element_type=jnp.float32, precision=hp)
    h2 = jnp.maximum(h2 + b2[...], 0.0)

    dt = dt_ref[...]
    s1 = jnp.sin(dt)
    c1 = jnp.cos(dt)
    s2 = 2.0 * s1 * c1
    c2 = 1.0 - 2.0 * s1 * s1
    s4 = 2.0 * s2 * c2
    c4 = 1.0 - 2.0 * s2 * s2
    s8 = 2.0 * s4 * c4
    c8 = 1.0 - 2.0 * s4 * s4
    zeros5 = jnp.zeros((5, dt.shape[1]), jnp.float32)
    pe48 = jnp.concatenate(
        [dt, s1, c1, s2, c2, s4, c4, s8, c8, zeros5, h2], axis=0)

    z = jnp.dot(w3t[...], pe48, preferred_element_type=jnp.float32,
                precision=hp)
    z = jnp.maximum(z + b3[...], 0.0)
    z = jnp.dot(w4t[...], z, preferred_element_type=jnp.float32, precision=hp)
    z = jnp.maximum(z + b4[...], 0.0)
    cm = jax.nn.sigmoid(
        jnp.dot(w5t[...], z, preferred_element_type=jnp.float32, precision=hp)
        + b5[...])

    xa = jnp.abs(xt_ref[...]) < 1.0
    m = xa[0:1, :] & xa[1:2, :] & xa[2:3, :]
    sigt_ref[...] = jnp.where(m, h2[15:16, :], 0.0)
    ct_ref[...] = jnp.where(m, cm, 0.0)


def _row2(i):
    return (i, 0)


def _col2(i):
    return (0, i)


def _fixed_spec(a):
    return pl.BlockSpec(a.shape, lambda i: (0,) * a.ndim)


def _mlp(feat, xt, dt, w1t, b1, w2t, b2, w3t, b3, w4t, b4, w5t, b5):
    weights = [w1t, b1, w2t, b2, w3t, b3, w4t, b4, w5t, b5]
    grid = (_B // _TB,)
    return pl.pallas_call(
        _mlp_body,
        grid=grid,
        in_specs=[
            pl.BlockSpec((_TB, _F), _row2),
            pl.BlockSpec((3, _TB), _col2),
            pl.BlockSpec((3, _TB), _col2),
        ] + [_fixed_spec(w) for w in weights],
        out_specs=[
            pl.BlockSpec((3, _TB), _col2),
            pl.BlockSpec((1, _TB), _col2),
        ],
        out_shape=[
            jax.ShapeDtypeStruct((3, _B), jnp.float32),
            jax.ShapeDtypeStruct((1, _B), jnp.float32),
        ],
    )(feat, xt, dt, *weights)


def kernel(x, d, xy_plane, yz_plane, xz_plane, W1, b1, W2, b2, W3, b3, W4,
           b4, W5, b5):
    x0 = x[:, 0]
    x1 = x[:, 1]
    x2 = x[:, 2]
    xyf = xy_plane.reshape(_N * _M, _F)
    yzf = yz_plane.reshape(_N * _M, _F)
    xzf = xz_plane.reshape(_N * _M, _F)
    feat = _feat(x0, x1, x2, xyf, yzf, xzf)

    w3full = jnp.concatenate(
        [W3[:27], jnp.zeros((5, _HID), jnp.float32), W3[27:],
         jnp.zeros((1, _HID), jnp.float32)], axis=0)
    ct, sigt = _mlp(feat, x.T, d.T, W1.T, b1.reshape(-1, 1), W2.T,
                    b2.reshape(-1, 1), w3full.T, b3.reshape(-1, 1), W4.T,
                    b4.reshape(-1, 1), W5.T, b5.reshape(-1, 1))
    return ct.T, sigt[0]
